# Initial kernel scaffold; baseline (speedup 1.0000x reference)
import jax
import jax.numpy as jnp
from jax.experimental import pallas as pl


def kernel(x, edge_index, W1, b1, W2, b2, W3, b3):
    raise NotImplementedError("write your pallas kernel here")


# speedup vs baseline: 14.3981x; 14.3981x over previous
import functools

import jax
import jax.numpy as jnp
from jax import lax
from jax.experimental import pallas as pl
from jax.experimental.pallas import tpu as pltpu
from jax.experimental.pallas import tpu_sc as plsc

N = 10000
NPAD = 10240
NC, NS = 2, 16
NW = NC * NS
K = 128
RPT = NPAD // NS


def _agg_body(F, y_hbm, srcs_hbm, dsts_hbm, out_hbm,
              src_v, dst_v, rows_v, zbuf, acc_sh, sem):
    chunks = srcs_hbm.shape[1]
    cid = lax.axis_index("c")
    sid = lax.axis_index("s")
    slab = cid * NS + sid

    groups = F // 16

    def zfill(i, _):
        r = i // groups
        g = i - r * groups
        zbuf[r, pl.ds(g * 16, 16)] = jnp.zeros((16,), jnp.float32)
        return 0

    lax.fori_loop(0, RPT * groups, zfill, 0)
    pltpu.sync_copy(zbuf, acc_sh.at[pl.ds(sid * RPT, RPT)])
    plsc.subcore_barrier()

    def chunk(ci, _):
        pltpu.sync_copy(srcs_hbm.at[slab, ci], src_v)
        pltpu.sync_copy(dsts_hbm.at[slab, ci], dst_v)
        pltpu.async_copy(y_hbm.at[src_v], rows_v, sem).wait()
        pltpu.sync_copy(rows_v, acc_sh.at[dst_v], add=True)
        return 0

    lax.fori_loop(0, chunks, chunk, 0)
    plsc.subcore_barrier()

    pltpu.sync_copy(acc_sh.at[pl.ds(sid * RPT, RPT)], zbuf)
    pltpu.sync_copy(zbuf, out_hbm.at[cid, pl.ds(sid * RPT, RPT)])


def _make_agg(F, chunks):
    mesh = plsc.VectorSubcoreMesh(core_axis_name="c", subcore_axis_name="s",
                                  num_cores=NC, num_subcores=NS)
    return pl.kernel(
        functools.partial(_agg_body, F),
        out_type=jax.ShapeDtypeStruct((NC, NPAD, F), jnp.float32),
        mesh=mesh,
        compiler_params=pltpu.CompilerParams(use_tc_tiling_on_sc=False),
        scratch_types=[
            pltpu.VMEM((K,), jnp.int32),
            pltpu.VMEM((K,), jnp.int32),
            pltpu.VMEM((K, F), jnp.float32),
            pltpu.VMEM((RPT, F), jnp.float32),
            pltpu.VMEM_SHARED((NPAD, F), jnp.float32),
            pltpu.SemaphoreType.DMA,
        ],
    )


def _hist_body(dsts_hbm, out_hbm, dst_v, ones_v, zbuf, acc_sh, sem):
    chunks = dsts_hbm.shape[1]
    cid = lax.axis_index("c")
    sid = lax.axis_index("s")
    slab = cid * NS + sid

    def ofill(i, _):
        ones_v[pl.ds(i * 16, 16)] = jnp.ones((16,), jnp.float32)
        return 0

    lax.fori_loop(0, K // 16, ofill, 0)

    def zfill(i, _):
        zbuf[pl.ds(i * 16, 16)] = jnp.zeros((16,), jnp.float32)
        return 0

    lax.fori_loop(0, RPT // 16, zfill, 0)
    pltpu.sync_copy(zbuf, acc_sh.at[pl.ds(sid * RPT, RPT)])
    plsc.subcore_barrier()

    def chunk(ci, _):
        pltpu.sync_copy(dsts_hbm.at[slab, ci], dst_v)
        pltpu.sync_copy(ones_v, acc_sh.at[dst_v], add=True)
        return 0

    lax.fori_loop(0, chunks, chunk, 0)
    plsc.subcore_barrier()

    pltpu.sync_copy(acc_sh.at[pl.ds(sid * RPT, RPT)], zbuf)
    pltpu.sync_copy(zbuf, out_hbm.at[cid, pl.ds(sid * RPT, RPT)])


def _make_hist(chunks):
    mesh = plsc.VectorSubcoreMesh(core_axis_name="c", subcore_axis_name="s",
                                  num_cores=NC, num_subcores=NS)
    return pl.kernel(
        _hist_body,
        out_type=jax.ShapeDtypeStruct((NC, NPAD), jnp.float32),
        mesh=mesh,
        compiler_params=pltpu.CompilerParams(use_tc_tiling_on_sc=False),
        scratch_types=[
            pltpu.VMEM((K,), jnp.int32),
            pltpu.VMEM((K,), jnp.float32),
            pltpu.VMEM((RPT,), jnp.float32),
            pltpu.VMEM_SHARED((NPAD,), jnp.float32),
            pltpu.SemaphoreType.DMA,
        ],
    )


_R = 1024


def _dis(dcol_ref):
    deg = dcol_ref[:, 0:1] + dcol_ref[:, 1:2] + 1.0
    return lax.rsqrt(deg)


def _tc_pre_body(x_ref, w_ref, dcol_ref, y_ref):
    dis = _dis(dcol_ref)
    y_ref[...] = jnp.dot(x_ref[...], w_ref[...],
                         preferred_element_type=jnp.float32) * dis


def _tc_pre(x, w, dcol):
    din = x.shape[1]
    fo = w.shape[1]
    grid = NPAD // _R
    return pl.pallas_call(
        _tc_pre_body,
        grid=(grid,),
        in_specs=[
            pl.BlockSpec((_R, din), lambda i: (i, 0)),
            pl.BlockSpec((din, fo), lambda i: (0, 0)),
            pl.BlockSpec((_R, 2), lambda i: (i, 0)),
        ],
        out_specs=pl.BlockSpec((_R, fo), lambda i: (i, 0)),
        out_shape=jax.ShapeDtypeStruct((NPAD, fo), jnp.float32),
    )(x, w, dcol)


def _tc_mid_body(acc_ref, yprev_ref, dcol_ref, w_ref, b_ref, out_ref):
    dis = _dis(dcol_ref)
    agg = acc_ref[0] + acc_ref[1] + yprev_ref[...]
    h = jnp.maximum(agg * dis + b_ref[...], 0.0)
    out_ref[...] = jnp.dot(h, w_ref[...],
                           preferred_element_type=jnp.float32) * dis


def _tc_mid(acc, yprev, dcol, w, b):
    fi = yprev.shape[1]
    fo = w.shape[1]
    grid = NPAD // _R
    return pl.pallas_call(
        _tc_mid_body,
        grid=(grid,),
        in_specs=[
            pl.BlockSpec((NC, _R, fi), lambda i: (0, i, 0)),
            pl.BlockSpec((_R, fi), lambda i: (i, 0)),
            pl.BlockSpec((_R, 2), lambda i: (i, 0)),
            pl.BlockSpec((fi, fo), lambda i: (0, 0)),
            pl.BlockSpec((1, fi), lambda i: (0, 0)),
        ],
        out_specs=pl.BlockSpec((_R, fo), lambda i: (i, 0)),
        out_shape=jax.ShapeDtypeStruct((NPAD, fo), jnp.float32),
    )(acc, yprev, dcol, w, b)


def _tc_post_body(acc_ref, yprev_ref, dcol_ref, b_ref, out_ref):
    dis = _dis(dcol_ref)
    agg = acc_ref[0] + acc_ref[1] + yprev_ref[...]
    out_ref[...] = agg * dis + b_ref[...]


def _tc_post(acc, yprev, dcol, b):
    fi = yprev.shape[1]
    grid = NPAD // _R
    return pl.pallas_call(
        _tc_post_body,
        grid=(grid,),
        in_specs=[
            pl.BlockSpec((NC, _R, fi), lambda i: (0, i, 0)),
            pl.BlockSpec((_R, fi), lambda i: (i, 0)),
            pl.BlockSpec((_R, 2), lambda i: (i, 0)),
            pl.BlockSpec((1, fi), lambda i: (0, 0)),
        ],
        out_specs=pl.BlockSpec((_R, fi), lambda i: (i, 0)),
        out_shape=jax.ShapeDtypeStruct((NPAD, fi), jnp.float32),
    )(acc, yprev, dcol, b)


def kernel(x, edge_index, W1, b1, W2, b2, W3, b3):
    E = edge_index.shape[1]
    epad = NW * K * (-(-E // (NW * K)))
    chunks = epad // (NW * K)

    ei = edge_index.astype(jnp.int32)
    npadrows = NPAD - N
    padv = N + (jnp.arange(epad - E, dtype=jnp.int32) % npadrows)
    src = jnp.concatenate([ei[0], padv]).reshape(NW, chunks, K)
    dst = jnp.concatenate([ei[1], padv]).reshape(NW, chunks, K)

    x_pad = jnp.pad(x, ((0, NPAD - N), (0, 0)))
    w3p = jnp.pad(W3, ((0, 0), (0, 6)))
    b3p = jnp.pad(b3, (0, 6))

    deg2 = _make_hist(chunks)(dst)
    dcol = jnp.transpose(deg2, (1, 0))

    y1 = _tc_pre(x_pad, W1, dcol)
    acc1 = _make_agg(32, chunks)(y1, src, dst)
    y2 = _tc_mid(acc1, y1, dcol, W2, b1.reshape(1, -1))
    acc2 = _make_agg(16, chunks)(y2, src, dst)
    y3 = _tc_mid(acc2, y2, dcol, w3p, b2.reshape(1, -1))
    acc3 = _make_agg(16, chunks)(y3, src, dst)
    out = _tc_post(acc3, y3, dcol, b3p.reshape(1, -1))
    return out[:N, :10]
